# baseline (speedup 1.0000x reference)
import functools

import jax
import jax.numpy as jnp
from jax import lax
from jax.experimental import pallas as pl
from jax.experimental.pallas import tpu as pltpu
from jax.experimental.pallas import tpu_sc as plsc

VOCAB = 100000
DIM = 64
NEG_K = 10
BATCH = 16384
CDF_LEN = 1024

NC, NS, L = 2, 16, 16
NW = NC * NS
RPW = BATCH // NW
SPW = RPW * NEG_K
NB = 8
BR = RPW // NB
GPR = 128 // L

PP_ROWS = BATCH // GPR
NP_ROWS = BATCH * NEG_K // GPR


def _cdf_body(counts_ref, cdf_ref):
    c = counts_ref[...]
    p = jnp.exp(0.75 * jnp.log(jnp.maximum(c, 1e-30)))
    p = jnp.where(c > 0.0, p, 0.0)
    p = p / jnp.sum(p)
    r = lax.broadcasted_iota(jnp.int32, (128, 128), 0)
    col = lax.broadcasted_iota(jnp.int32, (128, 128), 1)
    tri = (r <= col).astype(jnp.float32)
    rowcum = jnp.dot(p, tri, preferred_element_type=jnp.float32)
    rowsum = jnp.sum(p, axis=1, keepdims=True)
    ri = lax.broadcasted_iota(jnp.int32, (8, 8), 0)
    ci = lax.broadcasted_iota(jnp.int32, (8, 8), 1)
    strict = (ci < ri).astype(jnp.float32)
    off = jnp.dot(strict, rowsum, preferred_element_type=jnp.float32)
    cdf_ref[...] = rowcum + off


HALF = 50048
VPAD = 2 * HALF
_TBLK = 2176


def _detile_body(x1_ref, x2_ref, ident_ref, out_ref):
    dn = (((0,), (0,)), ((), ()))
    ident = ident_ref[...]
    xt1 = lax.dot_general(x1_ref[...], ident, dn,
                          preferred_element_type=jnp.float32)
    xt2 = lax.dot_general(x2_ref[...], ident, dn,
                          preferred_element_type=jnp.float32)
    out_ref[...] = jnp.concatenate([xt1, xt2], axis=1)


def _detile(table, ident):
    grid = HALF // _TBLK
    off = HALF // _TBLK
    pk = pl.pallas_call(
        _detile_body,
        grid=(grid,),
        in_specs=[
            pl.BlockSpec((DIM, _TBLK), lambda i: (0, i)),
            pl.BlockSpec((DIM, _TBLK), lambda i, o=off: (0, i + o)),
            pl.BlockSpec((DIM, DIM), lambda i: (0, 0)),
        ],
        out_specs=pl.BlockSpec((_TBLK, 128), lambda i: (i, 0)),
        out_shape=jax.ShapeDtypeStruct((HALF, 128), jnp.float32),
    )(table.T, table.T, ident)
    return pk.reshape(VPAD, DIM)


def _hash_u32(x):
    x = x ^ (x >> jnp.uint32(16))
    x = x * jnp.uint32(0x85EBCA6B)
    x = x ^ (x >> jnp.uint32(13))
    x = x * jnp.uint32(0xC2B2AE35)
    x = x ^ (x >> jnp.uint32(16))
    return x


def _sca_body(centers_hbm, cdf_hbm, ein_hbm, vcout_hbm, negout_hbm,
              cdf_v, cidx_v, negidx_v, vc_v, sem0):
    cid = lax.axis_index("c")
    sid = lax.axis_index("s")
    wid = sid * NC + cid
    base = wid * RPW

    pltpu.sync_copy(cdf_hbm, cdf_v)
    pltpu.sync_copy(centers_hbm.at[pl.ds(base, RPW)], cidx_v)

    @plsc.parallel_loop(0, RPW // L, 1, unroll=2)
    def remap_idx(t):
        c = cidx_v[pl.ds(t * L, L)]
        cidx_v[pl.ds(t * L, L)] = 2 * c - jnp.where(c < HALF, 0, VPAD - 1)

    pend = [
        pltpu.async_copy(ein_hbm.at[cidx_v.at[pl.ds(j * 128, 128)]],
                         vc_v.at[pl.ds(j * 128, 128)], sem0)
        for j in range(RPW // 128)
    ]

    lanes = lax.iota(jnp.int32, L)

    @plsc.parallel_loop(0, SPW // L, 1, unroll=4)
    def sample_vec(t):
        ctr = (wid * SPW + t * L + lanes).astype(jnp.uint32)
        h = _hash_u32(ctr)
        frac = plsc.bitcast(h & jnp.uint32(0xFFFFFF), jnp.int32)
        u = frac.astype(jnp.float32) * (1.0 / 16777216.0)
        cnt = jnp.zeros((L,), jnp.int32)
        for step in (512, 256, 128, 64, 32, 16, 8, 4, 2, 1):
            m = cnt + step
            vals = plsc.load_gather(cdf_v, [m - 1])
            cnt = jnp.where(vals <= u, m, cnt)
        negidx_v[pl.ds(t * L, L)] = 2 * jnp.minimum(cnt, VOCAB - 1)

    pltpu.sync_copy(negidx_v, negout_hbm.at[pl.ds(base * NEG_K, SPW)])
    for c in pend:
        c.wait()
    pltpu.sync_copy(vc_v, vcout_hbm.at[pl.ds(base, RPW)])


def _scb_body(pos_hbm, eout_hbm, vcin_hbm, negin_hbm, pp_hbm, np_hbm,
              pidx_v, negidx_v, vc_v, uo_v, uk_v, pp_v, np_v, sem0, sem1):
    cid = lax.axis_index("c")
    sid = lax.axis_index("s")
    wid = sid * NC + cid
    base = wid * RPW
    sems = (sem0, sem1)

    pltpu.sync_copy(pos_hbm.at[pl.ds(base, RPW)], pidx_v)
    pltpu.sync_copy(negin_hbm.at[pl.ds(base * NEG_K, SPW)], negidx_v)

    @plsc.parallel_loop(0, RPW // L, 1, unroll=2)
    def remap_idx(t):
        q = pidx_v[pl.ds(t * L, L)]
        pidx_v[pl.ds(t * L, L)] = 2 * q - jnp.where(q < HALF, 0, VPAD - 1)

    def issue_blk(b, buf):
        cps = [
            pltpu.async_copy(vcin_hbm.at[pl.ds(base + b * BR, BR)],
                             vc_v.at[buf], sems[buf]),
            pltpu.async_copy(eout_hbm.at[pidx_v.at[pl.ds(b * BR, BR)]],
                             uo_v.at[buf], sems[buf]),
        ]
        cps += [
            pltpu.async_copy(
                eout_hbm.at[negidx_v.at[pl.ds((b * NEG_K + k) * BR, BR)]],
                uk_v.at[buf, pl.ds(k * BR, BR)], sems[buf])
            for k in range(NEG_K)
        ]
        return cps

    pend = [issue_blk(0, 0), issue_blk(1, 1)]

    for b in range(NB):
        buf = b % 2
        for c in pend[b]:
            c.wait()
        if b + 2 < NB:
            pend.append(issue_blk(b + 2, buf))

        @plsc.parallel_loop(0, BR, 1, unroll=4)
        def row_body(r):
            v0 = vc_v[buf, r, pl.ds(0 * L, L)]
            v1 = vc_v[buf, r, pl.ds(1 * L, L)]
            v2 = vc_v[buf, r, pl.ds(2 * L, L)]
            v3 = vc_v[buf, r, pl.ds(3 * L, L)]
            pr = r // GPR
            lo = (r % GPR) * L
            acc = (v0 * uo_v[buf, r, pl.ds(0 * L, L)]
                   + v1 * uo_v[buf, r, pl.ds(1 * L, L)]
                   + v2 * uo_v[buf, r, pl.ds(2 * L, L)]
                   + v3 * uo_v[buf, r, pl.ds(3 * L, L)])
            pp_v[pr, pl.ds(lo, L)] = acc
            for k in range(NEG_K):
                kr = k * BR + r
                a = (v0 * uk_v[buf, kr, pl.ds(0 * L, L)]
                     + v1 * uk_v[buf, kr, pl.ds(1 * L, L)]
                     + v2 * uk_v[buf, kr, pl.ds(2 * L, L)]
                     + v3 * uk_v[buf, kr, pl.ds(3 * L, L)])
                np_v[k * (BR // GPR) + pr, pl.ds(lo, L)] = a

        pltpu.sync_copy(
            pp_v, pp_hbm.at[pl.ds(wid * (RPW // GPR) + b * (BR // GPR),
                                  BR // GPR)])
        pltpu.sync_copy(
            np_v, np_hbm.at[pl.ds((wid * NB + b) * (BR * NEG_K // GPR),
                                  BR * NEG_K // GPR)])


_SC_MESH = plsc.VectorSubcoreMesh(core_axis_name="c", subcore_axis_name="s")
_SC_PARAMS = pltpu.CompilerParams(
    needs_layout_passes=False, use_tc_tiling_on_sc=False)

_sgns_sca = functools.partial(
    pl.kernel,
    out_type=[
        jax.ShapeDtypeStruct((BATCH, DIM), jnp.float32),
        jax.ShapeDtypeStruct((BATCH * NEG_K,), jnp.int32),
    ],
    mesh=_SC_MESH,
    scratch_types=[
        pltpu.VMEM((CDF_LEN,), jnp.float32),
        pltpu.VMEM((RPW,), jnp.int32),
        pltpu.VMEM((SPW,), jnp.int32),
        pltpu.VMEM((RPW, DIM), jnp.float32),
        pltpu.SemaphoreType.DMA,
    ],
    compiler_params=_SC_PARAMS,
)(_sca_body)

_sgns_scb = functools.partial(
    pl.kernel,
    out_type=[
        jax.ShapeDtypeStruct((PP_ROWS, 128), jnp.float32),
        jax.ShapeDtypeStruct((NP_ROWS, 128), jnp.float32),
    ],
    mesh=_SC_MESH,
    scratch_types=[
        pltpu.VMEM((RPW,), jnp.int32),
        pltpu.VMEM((SPW,), jnp.int32),
        pltpu.VMEM((2, BR, DIM), jnp.float32),
        pltpu.VMEM((2, BR, DIM), jnp.float32),
        pltpu.VMEM((2, BR * NEG_K, DIM), jnp.float32),
        pltpu.VMEM((BR // GPR, 128), jnp.float32),
        pltpu.VMEM((BR * NEG_K // GPR, 128), jnp.float32),
        pltpu.SemaphoreType.DMA,
        pltpu.SemaphoreType.DMA,
    ],
    compiler_params=_SC_PARAMS,
)(_scb_body)


def _loss_body(pp_ref, np_ref, out_ref):
    i = pl.program_id(0)
    n = pl.num_programs(0)
    lane = lax.broadcasted_iota(jnp.int32, (128, GPR), 0)
    grp = lax.broadcasted_iota(jnp.int32, (128, GPR), 1)
    gmat = (lane // L == grp).astype(jnp.float32)

    def logsig(x):
        return jnp.minimum(x, 0.0) - jnp.log(1.0 + jnp.exp(-jnp.abs(x)))

    ps = jnp.dot(pp_ref[...], gmat, preferred_element_type=jnp.float32)
    ns = jnp.dot(np_ref[...], gmat, preferred_element_type=jnp.float32)
    partial = -jnp.sum(logsig(ps)) - jnp.sum(logsig(-ns))
    acc = jnp.where(i == 0, 0.0, out_ref[0, 0]) + partial
    out_ref[0, 0] = jnp.where(i == n - 1, acc / BATCH, acc)


@jax.jit
def kernel(centers, pos, embed_in, embed_out, counts):
    counts_p = jnp.pad(counts.astype(jnp.float32),
                       (0, CDF_LEN - counts.shape[0])).reshape(8, 128)
    cdf8 = pl.pallas_call(
        _cdf_body,
        out_shape=jax.ShapeDtypeStruct((8, 128), jnp.float32),
    )(counts_p)
    cdf = cdf8.reshape(CDF_LEN)

    ident = jnp.eye(DIM, dtype=jnp.float32)
    vc_rows, negidx = _sgns_sca(centers.astype(jnp.int32), cdf,
                                _detile(embed_in, ident))
    pp, npart = _sgns_scb(pos.astype(jnp.int32), _detile(embed_out, ident),
                          vc_rows, negidx)

    grid = 8
    loss = pl.pallas_call(
        _loss_body,
        grid=(grid,),
        in_specs=[
            pl.BlockSpec((PP_ROWS // grid, 128), lambda i: (i, 0)),
            pl.BlockSpec((NP_ROWS // grid, 128), lambda i: (i, 0)),
        ],
        out_specs=pl.BlockSpec(
            block_shape=(1, 1), index_map=lambda i: (0, 0),
            memory_space=pltpu.SMEM),
        out_shape=jax.ShapeDtypeStruct((1, 1), jnp.float32),
    )(pp, npart)
    return loss[0, 0]


# speedup vs baseline: 1.0139x; 1.0139x over previous
import functools

import jax
import jax.numpy as jnp
from jax import lax
from jax.experimental import pallas as pl
from jax.experimental.pallas import tpu as pltpu
from jax.experimental.pallas import tpu_sc as plsc

VOCAB = 100000
DIM = 64
NEG_K = 10
BATCH = 16384
CDF_LEN = 1024

NC, NS, L = 2, 16, 16
NW = NC * NS
RPW = BATCH // NW
SPW = RPW * NEG_K
NB = 8
BR = RPW // NB
GPR = 128 // L

PP_ROWS = BATCH // GPR
NP_ROWS = BATCH * NEG_K // GPR


def _cdf_body(counts_ref, cdf_ref):
    c = counts_ref[...]
    p = jnp.exp(0.75 * jnp.log(jnp.maximum(c, 1e-30)))
    p = jnp.where(c > 0.0, p, 0.0)
    p = p / jnp.sum(p)
    r = lax.broadcasted_iota(jnp.int32, (128, 128), 0)
    col = lax.broadcasted_iota(jnp.int32, (128, 128), 1)
    tri = (r <= col).astype(jnp.float32)
    rowcum = jnp.dot(p, tri, preferred_element_type=jnp.float32)
    rowsum = jnp.sum(p, axis=1, keepdims=True)
    ri = lax.broadcasted_iota(jnp.int32, (8, 8), 0)
    ci = lax.broadcasted_iota(jnp.int32, (8, 8), 1)
    strict = (ci < ri).astype(jnp.float32)
    off = jnp.dot(strict, rowsum, preferred_element_type=jnp.float32)
    cdf_ref[...] = rowcum + off


HALF = 50048
VPAD = 2 * HALF
_TBLK = 2176


def _detile_body(x1_ref, x2_ref, ident_ref, out_ref):
    dn = (((0,), (0,)), ((), ()))
    ident = ident_ref[...]
    xt1 = lax.dot_general(x1_ref[...], ident, dn,
                          preferred_element_type=jnp.float32)
    xt2 = lax.dot_general(x2_ref[...], ident, dn,
                          preferred_element_type=jnp.float32)
    out_ref[...] = jnp.concatenate([xt1, xt2], axis=1)


def _detile(table, ident):
    grid = HALF // _TBLK
    off = HALF // _TBLK
    pk = pl.pallas_call(
        _detile_body,
        grid=(grid,),
        in_specs=[
            pl.BlockSpec((DIM, _TBLK), lambda i: (0, i)),
            pl.BlockSpec((DIM, _TBLK), lambda i, o=off: (0, i + o)),
            pl.BlockSpec((DIM, DIM), lambda i: (0, 0)),
        ],
        out_specs=pl.BlockSpec((_TBLK, 128), lambda i: (i, 0)),
        out_shape=jax.ShapeDtypeStruct((HALF, 128), jnp.float32),
    )(table.T, table.T, ident)
    return pk.reshape(VPAD, DIM)


def _hash_u32(x):
    x = x ^ (x >> jnp.uint32(16))
    x = x * jnp.uint32(0x85EBCA6B)
    x = x ^ (x >> jnp.uint32(13))
    x = x * jnp.uint32(0xC2B2AE35)
    x = x ^ (x >> jnp.uint32(16))
    return x


def _sca_body(centers_hbm, cdf_hbm, ein_hbm, vcout_hbm, negout_hbm,
              cdf_v, cidx_v, negidx_v, vc_v, sem0):
    cid = lax.axis_index("c")
    sid = lax.axis_index("s")
    wid = sid * NC + cid
    base = wid * RPW

    pltpu.sync_copy(cdf_hbm, cdf_v)
    pltpu.sync_copy(centers_hbm.at[pl.ds(base, RPW)], cidx_v)

    @plsc.parallel_loop(0, RPW // L, 1, unroll=2)
    def remap_idx(t):
        c = cidx_v[pl.ds(t * L, L)]
        cidx_v[pl.ds(t * L, L)] = 2 * c - jnp.where(c < HALF, 0, VPAD - 1)

    pend = [
        pltpu.async_copy(ein_hbm.at[cidx_v.at[pl.ds(j * 128, 128)]],
                         vc_v.at[pl.ds(j * 128, 128)], sem0)
        for j in range(RPW // 128)
    ]

    lanes = lax.iota(jnp.int32, L)

    @plsc.parallel_loop(0, SPW // L, 1, unroll=4)
    def sample_vec(t):
        ctr = (wid * SPW + t * L + lanes).astype(jnp.uint32)
        h = _hash_u32(ctr)
        frac = plsc.bitcast(h & jnp.uint32(0xFFFFFF), jnp.int32)
        u = frac.astype(jnp.float32) * (1.0 / 16777216.0)
        cnt = jnp.zeros((L,), jnp.int32)
        for step in (512, 256, 128, 64, 32, 16, 8, 4, 2, 1):
            m = cnt + step
            vals = plsc.load_gather(cdf_v, [m - 1])
            cnt = jnp.where(vals <= u, m, cnt)
        negidx_v[pl.ds(t * L, L)] = 2 * jnp.minimum(cnt, VOCAB - 1)

    pltpu.sync_copy(negidx_v, negout_hbm.at[pl.ds(base * NEG_K, SPW)])
    for c in pend:
        c.wait()
    pltpu.sync_copy(vc_v, vcout_hbm.at[pl.ds(base, RPW)])


def _scb_body(pos_hbm, eout_hbm, vcin_hbm, negin_hbm, pp_hbm, np_hbm,
              pidx_v, negidx_v, vc_v, uo_v, uk_v, pp_v, np_v, sem0, sem1):
    cid = lax.axis_index("c")
    sid = lax.axis_index("s")
    wid = sid * NC + cid
    base = wid * RPW
    sems = (sem0, sem1)

    pltpu.sync_copy(pos_hbm.at[pl.ds(base, RPW)], pidx_v)
    pltpu.sync_copy(negin_hbm.at[pl.ds(base * NEG_K, SPW)], negidx_v)

    @plsc.parallel_loop(0, RPW // L, 1, unroll=2)
    def remap_idx(t):
        q = pidx_v[pl.ds(t * L, L)]
        pidx_v[pl.ds(t * L, L)] = 2 * q - jnp.where(q < HALF, 0, VPAD - 1)

    def issue_blk(b, buf):
        cps = [
            pltpu.async_copy(vcin_hbm.at[pl.ds(base + b * BR, BR)],
                             vc_v.at[buf], sems[buf]),
            pltpu.async_copy(eout_hbm.at[pidx_v.at[pl.ds(b * BR, BR)]],
                             uo_v.at[buf], sems[buf]),
        ]
        cps += [
            pltpu.async_copy(
                eout_hbm.at[negidx_v.at[pl.ds((b * NEG_K + k) * BR, BR)]],
                uk_v.at[buf, pl.ds(k * BR, BR)], sems[buf])
            for k in range(NEG_K)
        ]
        return cps

    pend = [issue_blk(0, 0), issue_blk(1, 1)]

    for b in range(NB):
        buf = b % 2
        for c in pend[b]:
            c.wait()
        if b + 2 < NB:
            pend.append(issue_blk(b + 2, buf))

        @plsc.parallel_loop(0, BR, 1, unroll=2)
        def row_body(r):
            v0 = vc_v[buf, r, pl.ds(0 * L, L)]
            v1 = vc_v[buf, r, pl.ds(1 * L, L)]
            v2 = vc_v[buf, r, pl.ds(2 * L, L)]
            v3 = vc_v[buf, r, pl.ds(3 * L, L)]
            pr = r // GPR
            lo = (r % GPR) * L
            acc = (v0 * uo_v[buf, r, pl.ds(0 * L, L)]
                   + v1 * uo_v[buf, r, pl.ds(1 * L, L)]
                   + v2 * uo_v[buf, r, pl.ds(2 * L, L)]
                   + v3 * uo_v[buf, r, pl.ds(3 * L, L)])
            pp_v[pr, pl.ds(lo, L)] = acc
            for k in range(NEG_K):
                kr = k * BR + r
                a = (v0 * uk_v[buf, kr, pl.ds(0 * L, L)]
                     + v1 * uk_v[buf, kr, pl.ds(1 * L, L)]
                     + v2 * uk_v[buf, kr, pl.ds(2 * L, L)]
                     + v3 * uk_v[buf, kr, pl.ds(3 * L, L)])
                np_v[k * (BR // GPR) + pr, pl.ds(lo, L)] = a

        pltpu.sync_copy(
            pp_v, pp_hbm.at[pl.ds(wid * (RPW // GPR) + b * (BR // GPR),
                                  BR // GPR)])
        pltpu.sync_copy(
            np_v, np_hbm.at[pl.ds((wid * NB + b) * (BR * NEG_K // GPR),
                                  BR * NEG_K // GPR)])


_SC_MESH = plsc.VectorSubcoreMesh(core_axis_name="c", subcore_axis_name="s")
_SC_PARAMS = pltpu.CompilerParams(
    needs_layout_passes=False, use_tc_tiling_on_sc=False)

_sgns_sca = functools.partial(
    pl.kernel,
    out_type=[
        jax.ShapeDtypeStruct((BATCH, DIM), jnp.float32),
        jax.ShapeDtypeStruct((BATCH * NEG_K,), jnp.int32),
    ],
    mesh=_SC_MESH,
    scratch_types=[
        pltpu.VMEM((CDF_LEN,), jnp.float32),
        pltpu.VMEM((RPW,), jnp.int32),
        pltpu.VMEM((SPW,), jnp.int32),
        pltpu.VMEM((RPW, DIM), jnp.float32),
        pltpu.SemaphoreType.DMA,
    ],
    compiler_params=_SC_PARAMS,
)(_sca_body)

_sgns_scb = functools.partial(
    pl.kernel,
    out_type=[
        jax.ShapeDtypeStruct((PP_ROWS, 128), jnp.float32),
        jax.ShapeDtypeStruct((NP_ROWS, 128), jnp.float32),
    ],
    mesh=_SC_MESH,
    scratch_types=[
        pltpu.VMEM((RPW,), jnp.int32),
        pltpu.VMEM((SPW,), jnp.int32),
        pltpu.VMEM((2, BR, DIM), jnp.float32),
        pltpu.VMEM((2, BR, DIM), jnp.float32),
        pltpu.VMEM((2, BR * NEG_K, DIM), jnp.float32),
        pltpu.VMEM((BR // GPR, 128), jnp.float32),
        pltpu.VMEM((BR * NEG_K // GPR, 128), jnp.float32),
        pltpu.SemaphoreType.DMA,
        pltpu.SemaphoreType.DMA,
    ],
    compiler_params=_SC_PARAMS,
)(_scb_body)


def _loss_body(pp_ref, np_ref, out_ref):
    i = pl.program_id(0)
    n = pl.num_programs(0)
    lane = lax.broadcasted_iota(jnp.int32, (128, GPR), 0)
    grp = lax.broadcasted_iota(jnp.int32, (128, GPR), 1)
    gmat = (lane // L == grp).astype(jnp.float32)

    def logsig(x):
        return jnp.minimum(x, 0.0) - jnp.log(1.0 + jnp.exp(-jnp.abs(x)))

    ps = jnp.dot(pp_ref[...], gmat, preferred_element_type=jnp.float32)
    ns = jnp.dot(np_ref[...], gmat, preferred_element_type=jnp.float32)
    partial = -jnp.sum(logsig(ps)) - jnp.sum(logsig(-ns))
    acc = jnp.where(i == 0, 0.0, out_ref[0, 0]) + partial
    out_ref[0, 0] = jnp.where(i == n - 1, acc / BATCH, acc)


@jax.jit
def kernel(centers, pos, embed_in, embed_out, counts):
    counts_p = jnp.pad(counts.astype(jnp.float32),
                       (0, CDF_LEN - counts.shape[0])).reshape(8, 128)
    cdf8 = pl.pallas_call(
        _cdf_body,
        out_shape=jax.ShapeDtypeStruct((8, 128), jnp.float32),
    )(counts_p)
    cdf = cdf8.reshape(CDF_LEN)

    ident = jnp.eye(DIM, dtype=jnp.float32)
    vc_rows, negidx = _sgns_sca(centers.astype(jnp.int32), cdf,
                                _detile(embed_in, ident))
    pp, npart = _sgns_scb(pos.astype(jnp.int32), _detile(embed_out, ident),
                          vc_rows, negidx)

    grid = 8
    loss = pl.pallas_call(
        _loss_body,
        grid=(grid,),
        in_specs=[
            pl.BlockSpec((PP_ROWS // grid, 128), lambda i: (i, 0)),
            pl.BlockSpec((NP_ROWS // grid, 128), lambda i: (i, 0)),
        ],
        out_specs=pl.BlockSpec(
            block_shape=(1, 1), index_map=lambda i: (0, 0),
            memory_space=pltpu.SMEM),
        out_shape=jax.ShapeDtypeStruct((1, 1), jnp.float32),
    )(pp, npart)
    return loss[0, 0]
